# baseline (speedup 1.0000x reference)
import functools

import numpy as np

import jax
import jax.numpy as jnp
from jax import lax
from jax.experimental import pallas as pl
from jax.experimental.pallas import tpu as pltpu
from jax.experimental.pallas import tpu_sc as plsc

N_NODES = 10000
K = 32
D = 128
DW = D // 2

NC = 2
NS = 16
NW = NC * NS

NPAD = 10240
N0 = 3072
N1 = NPAD - N0
PER0 = N0 // NS
PER1 = N1 // NS
PER_MAX = max(PER0, PER1)
CHUNK = 4
NBUF = 4
GROUP = CHUNK * NBUF
G0 = PER0 // GROUP
G1 = PER1 // GROUP
IDX_MAX = PER_MAX * K // D
NTBL = NPAD + 512
NIDX_ROWS = NPAD * K // D + IDX_MAX


def _mm_body(x_ref, w_ref, z_ref):
    z_ref[...] = lax.dot_general(
        x_ref[...], w_ref[...],
        dimension_numbers=(((1,), (1,)), ((), ())),
        preferred_element_type=jnp.float32,
    ).astype(jnp.bfloat16)


def _matmul_bf16(x_pad, wp):
    blk = 448
    grid = NTBL // blk
    return pl.pallas_call(
        _mm_body,
        grid=(grid,),
        in_specs=[
            pl.BlockSpec((blk, D), lambda i: (i, 0)),
            pl.BlockSpec((D, D), lambda i: (0, 0)),
        ],
        out_specs=pl.BlockSpec((blk, D), lambda i: (i, 0)),
        out_shape=jax.ShapeDtypeStruct((NTBL, D), jnp.bfloat16),
    )(x_pad, wp)


_PERM = np.empty((D,), dtype=np.int32)
for _p in range(D // 32):
    for _k in range(16):
        _PERM[32 * _p + 2 * _k] = 32 * _p + _k
        _PERM[32 * _p + 2 * _k + 1] = 32 * _p + 16 + _k


def _unpack2(w):
    lo = lax.bitcast_convert_type(lax.shift_left(w, 16), jnp.float32)
    hi = lax.bitcast_convert_type(lax.bitwise_and(w, jnp.int32(-65536)),
                                  jnp.float32)
    return lo, hi


def _reduce_chunk(gat, own_v, out_v, t, urow):
    inv_k = jnp.float32(1.0 / K)
    zero = jnp.float32(0.0)
    for n in range(CHUNK):
        for p in range(DW // 16):
            sl = pl.ds(p * 16, 16)
            acc_lo, acc_hi = _unpack2(gat[n * K, sl])
            for j in range(1, K):
                lo, hi = _unpack2(gat[n * K + j, sl])
                acc_lo = acc_lo + lo
                acc_hi = acc_hi + hi
            own_lo, own_hi = _unpack2(own_v[t * CHUNK + n, sl])
            out_v[urow + n, pl.ds(32 * p, 16)] = jnp.maximum(
                own_lo - acc_lo * inv_k, zero)
            out_v[urow + n, pl.ds(32 * p + 16, 16)] = jnp.maximum(
                own_hi - acc_hi * inv_k, zero)


def _sc_body(z_hbm, nidx_hbm, out_hbm,
             idx_v, g0, g1, g2, g3, own_v, u_all,
             s0, s1, s2, s3, su, sem_own):
    c = lax.axis_index("c")
    s = lax.axis_index("s")
    node_base = jnp.where(c == 0, s * PER0, N0 + s * PER1)
    n_groups = jnp.where(c == 0, G0, G1)
    n_chunks = n_groups * NBUF
    gbufs = (g0, g1, g2, g3)
    gsems = (s0, s1, s2, s3)

    own_cp = pltpu.async_copy(z_hbm.at[pl.ds(node_base, PER_MAX)], own_v,
                              sem_own)
    pltpu.sync_copy(nidx_hbm.at[pl.ds(node_base * K // D, IDX_MAX)], idx_v)

    def gather(t, g, sem):
        tc = jnp.minimum(t, n_chunks - 1)
        pltpu.async_copy(z_hbm.at[idx_v.at[tc]], g, sem)

    def wait_gather(g, sem):
        pltpu.make_async_copy(z_hbm.at[pl.ds(0, D)], g, sem).wait()

    for b in range(NBUF):
        gather(b, gbufs[b], gsems[b])
    own_cp.wait()

    def loop_body(grp, _):
        t0 = NBUF * grp
        @pl.when(grp >= 1)
        def _drain():
            pltpu.make_async_copy(
                u_all, out_hbm.at[pl.ds(0, GROUP)], su).wait()
        for b in range(NBUF):
            t = t0 + b
            wait_gather(gbufs[b], gsems[b])
            _reduce_chunk(gbufs[b], own_v, u_all, t, b * CHUNK)
            gather(t + NBUF, gbufs[b], gsems[b])
        pltpu.async_copy(
            u_all, out_hbm.at[pl.ds(node_base + t0 * CHUNK, GROUP)], su)
        return _

    lax.fori_loop(0, n_groups, loop_body, None)
    for b in range(NBUF):
        wait_gather(gbufs[b], gsems[b])
    pltpu.make_async_copy(u_all, out_hbm.at[pl.ds(0, GROUP)], su).wait()


_sc_call = functools.partial(
    pl.kernel,
    out_type=jax.ShapeDtypeStruct((NPAD, D), jnp.float32),
    mesh=plsc.VectorSubcoreMesh(core_axis_name="c", subcore_axis_name="s"),
    compiler_params=pltpu.CompilerParams(use_tc_tiling_on_sc=False),
    scratch_types=[
        pltpu.VMEM((IDX_MAX, D), jnp.int32),
        pltpu.VMEM((D, DW), jnp.int32),
        pltpu.VMEM((D, DW), jnp.int32),
        pltpu.VMEM((D, DW), jnp.int32),
        pltpu.VMEM((D, DW), jnp.int32),
        pltpu.VMEM((PER_MAX, DW), jnp.int32),
        pltpu.VMEM((GROUP, D), jnp.float32),
        pltpu.SemaphoreType.DMA,
        pltpu.SemaphoreType.DMA,
        pltpu.SemaphoreType.DMA,
        pltpu.SemaphoreType.DMA,
        pltpu.SemaphoreType.DMA,
        pltpu.SemaphoreType.DMA,
    ],
)(_sc_body)


def kernel(X, neigh_idx, W):
    x_pad = jnp.zeros((NTBL, D), jnp.float32).at[:N_NODES].set(X)
    nidx_pad = jnp.zeros((NIDX_ROWS * D // K, K), jnp.int32)
    nidx_pad = nidx_pad.at[:N_NODES].set(neigh_idx)
    nidx2d = nidx_pad.reshape(NIDX_ROWS, D)
    wp = W[jnp.asarray(_PERM)]
    zb = _matmul_bf16(x_pad, wp)
    zi = lax.bitcast_convert_type(zb.reshape(NTBL, DW, 2), jnp.int32)
    out = _sc_call(zi, nidx2d)
    return out[:N_NODES]


# speedup vs baseline: 1.6691x; 1.6691x over previous
import functools

import numpy as np

import jax
import jax.numpy as jnp
from jax import lax
from jax.experimental import pallas as pl
from jax.experimental.pallas import tpu as pltpu
from jax.experimental.pallas import tpu_sc as plsc

N_NODES = 10000
K = 32
D = 128
DW = D // 2

NC = 2
NS = 16
NW = NC * NS

NPAD = 10240
N0 = 7168
N1 = NPAD - N0
PER0 = N0 // NS
PER1 = N1 // NS
PER_MAX = max(PER0, PER1)
CHUNK = 4
NBUF = 4
GROUP = CHUNK * NBUF
G0 = PER0 // GROUP
G1 = PER1 // GROUP
IDX_MAX = PER_MAX * K // D
NTBL = NPAD + 512
NIDX_ROWS = NPAD * K // D + IDX_MAX


def _mm_body(x_ref, w_ref, z_ref):
    z_ref[...] = lax.dot_general(
        x_ref[...], w_ref[...],
        dimension_numbers=(((1,), (1,)), ((), ())),
        preferred_element_type=jnp.float32,
    ).astype(jnp.bfloat16)


def _matmul_bf16(x_pad, wp):
    blk = 448
    grid = NTBL // blk
    return pl.pallas_call(
        _mm_body,
        grid=(grid,),
        in_specs=[
            pl.BlockSpec((blk, D), lambda i: (i, 0)),
            pl.BlockSpec((D, D), lambda i: (0, 0)),
        ],
        out_specs=pl.BlockSpec((blk, D), lambda i: (i, 0)),
        out_shape=jax.ShapeDtypeStruct((NTBL, D), jnp.bfloat16),
    )(x_pad, wp)


_PERM = np.empty((D,), dtype=np.int32)
for _p in range(D // 32):
    for _k in range(16):
        _PERM[32 * _p + 2 * _k] = 32 * _p + _k
        _PERM[32 * _p + 2 * _k + 1] = 32 * _p + 16 + _k


def _unpack2(w):
    lo = lax.bitcast_convert_type(lax.shift_left(w, 16), jnp.float32)
    hi = lax.bitcast_convert_type(lax.bitwise_and(w, jnp.int32(-65536)),
                                  jnp.float32)
    return lo, hi


def _reduce_chunk(gat, own_v, out_v, t, urow):
    inv_k = jnp.float32(1.0 / K)
    zero = jnp.float32(0.0)
    for n in range(CHUNK):
        for p in range(DW // 16):
            sl = pl.ds(p * 16, 16)
            acc_lo, acc_hi = _unpack2(gat[n * K, sl])
            for j in range(1, K):
                lo, hi = _unpack2(gat[n * K + j, sl])
                acc_lo = acc_lo + lo
                acc_hi = acc_hi + hi
            own_lo, own_hi = _unpack2(own_v[t * CHUNK + n, sl])
            out_v[urow + n, pl.ds(32 * p, 16)] = jnp.maximum(
                own_lo - acc_lo * inv_k, zero)
            out_v[urow + n, pl.ds(32 * p + 16, 16)] = jnp.maximum(
                own_hi - acc_hi * inv_k, zero)


def _sc_body(z_hbm, nidx_hbm, out_hbm,
             idx_v, g0, g1, g2, g3, own_v, u_all,
             s0, s1, s2, s3, su, sem_own):
    c = lax.axis_index("c")
    s = lax.axis_index("s")
    node_base = jnp.where(c == 0, s * PER0, N0 + s * PER1)
    n_groups = jnp.where(c == 0, G0, G1)
    n_chunks = n_groups * NBUF
    gbufs = (g0, g1, g2, g3)
    gsems = (s0, s1, s2, s3)

    own_cp = pltpu.async_copy(z_hbm.at[pl.ds(node_base, PER_MAX)], own_v,
                              sem_own)
    pltpu.sync_copy(nidx_hbm.at[pl.ds(node_base * K // D, IDX_MAX)], idx_v)

    def gather(t, g, sem):
        tc = jnp.minimum(t, n_chunks - 1)
        pltpu.async_copy(z_hbm.at[idx_v.at[tc]], g, sem)

    def wait_gather(g, sem):
        pltpu.make_async_copy(z_hbm.at[pl.ds(0, D)], g, sem).wait()

    for b in range(NBUF):
        gather(b, gbufs[b], gsems[b])
    own_cp.wait()

    def loop_body(grp, _):
        t0 = NBUF * grp
        @pl.when(grp >= 1)
        def _drain():
            pltpu.make_async_copy(
                u_all, out_hbm.at[pl.ds(0, GROUP)], su).wait()
        for b in range(NBUF):
            t = t0 + b
            wait_gather(gbufs[b], gsems[b])
            _reduce_chunk(gbufs[b], own_v, u_all, t, b * CHUNK)
            gather(t + NBUF, gbufs[b], gsems[b])
        pltpu.async_copy(
            u_all, out_hbm.at[pl.ds(node_base + t0 * CHUNK, GROUP)], su)
        return _

    lax.fori_loop(0, n_groups, loop_body, None)
    for b in range(NBUF):
        wait_gather(gbufs[b], gsems[b])
    pltpu.make_async_copy(u_all, out_hbm.at[pl.ds(0, GROUP)], su).wait()


_sc_call = functools.partial(
    pl.kernel,
    out_type=jax.ShapeDtypeStruct((NPAD, D), jnp.float32),
    mesh=plsc.VectorSubcoreMesh(core_axis_name="c", subcore_axis_name="s"),
    compiler_params=pltpu.CompilerParams(use_tc_tiling_on_sc=False),
    scratch_types=[
        pltpu.VMEM((IDX_MAX, D), jnp.int32),
        pltpu.VMEM((D, DW), jnp.int32),
        pltpu.VMEM((D, DW), jnp.int32),
        pltpu.VMEM((D, DW), jnp.int32),
        pltpu.VMEM((D, DW), jnp.int32),
        pltpu.VMEM((PER_MAX, DW), jnp.int32),
        pltpu.VMEM((GROUP, D), jnp.float32),
        pltpu.SemaphoreType.DMA,
        pltpu.SemaphoreType.DMA,
        pltpu.SemaphoreType.DMA,
        pltpu.SemaphoreType.DMA,
        pltpu.SemaphoreType.DMA,
        pltpu.SemaphoreType.DMA,
    ],
)(_sc_body)


def kernel(X, neigh_idx, W):
    x_pad = jnp.zeros((NTBL, D), jnp.float32).at[:N_NODES].set(X)
    nidx_pad = jnp.zeros((NIDX_ROWS * D // K, K), jnp.int32)
    nidx_pad = nidx_pad.at[:N_NODES].set(neigh_idx)
    nidx2d = nidx_pad.reshape(NIDX_ROWS, D)
    wp = W[jnp.asarray(_PERM)]
    zb = _matmul_bf16(x_pad, wp)
    zi = lax.bitcast_convert_type(zb.reshape(NTBL, DW, 2), jnp.int32)
    out = _sc_call(zi, nidx2d)
    return out[:N_NODES]


# speedup vs baseline: 1.6864x; 1.0103x over previous
import functools

import numpy as np

import jax
import jax.numpy as jnp
from jax import lax
from jax.experimental import pallas as pl
from jax.experimental.pallas import tpu as pltpu
from jax.experimental.pallas import tpu_sc as plsc

N_NODES = 10000
K = 32
D = 128
DW = D // 2

NC = 2
NS = 16
NW = NC * NS

N0 = 7168
PER0 = N0 // NS
PER1A = 176
PER1B = 192
PER_MAX = PER0
CHUNK = 4
NBUF = 4
GROUP = CHUNK * NBUF
G0 = PER0 // GROUP
G1A = PER1A // GROUP
G1B = PER1B // GROUP
IDX_MAX = PER_MAX * K // D
NTBL = 10752
NIDX_ROWS = N_NODES * K // D + IDX_MAX


def _mm_body(x_ref, w_ref, z_ref):
    z_ref[...] = lax.dot_general(
        x_ref[...], w_ref[...],
        dimension_numbers=(((1,), (1,)), ((), ())),
        preferred_element_type=jnp.float32,
    ).astype(jnp.bfloat16)


def _matmul_packed(x, wp):
    blk = 448
    grid = 23
    return pl.pallas_call(
        _mm_body,
        grid=(grid,),
        in_specs=[
            pl.BlockSpec((blk, D), lambda i: (i, 0)),
            pl.BlockSpec((D, D), lambda i: (0, 0)),
        ],
        out_specs=pl.BlockSpec((blk, D), lambda i: (i, 0)),
        out_shape=jax.ShapeDtypeStruct((NTBL, D), jnp.bfloat16),
    )(x, wp)


_PERM = np.empty((D,), dtype=np.int32)
for _p in range(D // 32):
    for _k in range(16):
        _PERM[32 * _p + 2 * _k] = 32 * _p + _k
        _PERM[32 * _p + 2 * _k + 1] = 32 * _p + 16 + _k


def _unpack2(w):
    lo = lax.bitcast_convert_type(lax.shift_left(w, 16), jnp.float32)
    hi = lax.bitcast_convert_type(lax.bitwise_and(w, jnp.int32(-65536)),
                                  jnp.float32)
    return lo, hi


def _reduce_chunk(gat, own_v, out_v, t, urow):
    inv_k = jnp.float32(1.0 / K)
    zero = jnp.float32(0.0)
    for n in range(CHUNK):
        for p in range(DW // 16):
            sl = pl.ds(p * 16, 16)
            acc_lo, acc_hi = _unpack2(gat[n * K, sl])
            for j in range(1, K):
                lo, hi = _unpack2(gat[n * K + j, sl])
                acc_lo = acc_lo + lo
                acc_hi = acc_hi + hi
            own_lo, own_hi = _unpack2(own_v[t * CHUNK + n, sl])
            out_v[urow + n, pl.ds(32 * p, 16)] = jnp.maximum(
                own_lo - acc_lo * inv_k, zero)
            out_v[urow + n, pl.ds(32 * p + 16, 16)] = jnp.maximum(
                own_hi - acc_hi * inv_k, zero)


def _sc_body(z_hbm, nidx_hbm, out_hbm,
             idx_v, g0, g1, g2, g3, own_v, u_all,
             s0, s1, s2, s3, su, sem_own):
    c = lax.axis_index("c")
    s = lax.axis_index("s")
    node_base = jnp.where(c == 0, s * PER0, N0 + s * PER1A)
    n_groups = jnp.where(c == 0, G0, jnp.where(s < NS - 1, G1A, G1B))
    n_chunks = n_groups * NBUF
    gbufs = (g0, g1, g2, g3)
    gsems = (s0, s1, s2, s3)

    own_cp = pltpu.async_copy(z_hbm.at[pl.ds(node_base, PER_MAX)], own_v,
                              sem_own)
    pltpu.sync_copy(nidx_hbm.at[pl.ds(node_base * K // D, IDX_MAX)], idx_v)

    def gather(t, g, sem):
        tc = jnp.minimum(t, n_chunks - 1)
        pltpu.async_copy(z_hbm.at[idx_v.at[tc]], g, sem)

    def wait_gather(g, sem):
        pltpu.make_async_copy(z_hbm.at[pl.ds(0, D)], g, sem).wait()

    for b in range(NBUF):
        gather(b, gbufs[b], gsems[b])
    own_cp.wait()

    def loop_body(grp, _):
        t0 = NBUF * grp
        @pl.when(grp >= 1)
        def _drain():
            pltpu.make_async_copy(
                u_all, out_hbm.at[pl.ds(0, GROUP)], su).wait()
        for b in range(NBUF):
            t = t0 + b
            wait_gather(gbufs[b], gsems[b])
            _reduce_chunk(gbufs[b], own_v, u_all, t, b * CHUNK)
            gather(t + NBUF, gbufs[b], gsems[b])
        pltpu.async_copy(
            u_all, out_hbm.at[pl.ds(node_base + t0 * CHUNK, GROUP)], su)
        return _

    lax.fori_loop(0, n_groups, loop_body, None)
    for b in range(NBUF):
        wait_gather(gbufs[b], gsems[b])
    pltpu.make_async_copy(u_all, out_hbm.at[pl.ds(0, GROUP)], su).wait()


_sc_call = functools.partial(
    pl.kernel,
    out_type=jax.ShapeDtypeStruct((N_NODES, D), jnp.float32),
    mesh=plsc.VectorSubcoreMesh(core_axis_name="c", subcore_axis_name="s"),
    compiler_params=pltpu.CompilerParams(use_tc_tiling_on_sc=False),
    scratch_types=[
        pltpu.VMEM((IDX_MAX, D), jnp.int32),
        pltpu.VMEM((D, DW), jnp.int32),
        pltpu.VMEM((D, DW), jnp.int32),
        pltpu.VMEM((D, DW), jnp.int32),
        pltpu.VMEM((D, DW), jnp.int32),
        pltpu.VMEM((PER_MAX, DW), jnp.int32),
        pltpu.VMEM((GROUP, D), jnp.float32),
        pltpu.SemaphoreType.DMA,
        pltpu.SemaphoreType.DMA,
        pltpu.SemaphoreType.DMA,
        pltpu.SemaphoreType.DMA,
        pltpu.SemaphoreType.DMA,
        pltpu.SemaphoreType.DMA,
    ],
)(_sc_body)


def kernel(X, neigh_idx, W):
    nidx2d = jnp.zeros((NIDX_ROWS, D), jnp.int32)
    nidx2d = nidx2d.at[:N_NODES * K // D].set(
        neigh_idx.reshape(N_NODES * K // D, D))
    wp = W[jnp.asarray(_PERM)]
    zb = _matmul_packed(X, wp)
    zi = lax.bitcast_convert_type(zb.reshape(NTBL, DW, 2), jnp.int32)
    return _sc_call(zi, nidx2d)


# speedup vs baseline: 1.7494x; 1.0374x over previous
import functools

import numpy as np

import jax
import jax.numpy as jnp
from jax import lax
from jax.experimental import pallas as pl
from jax.experimental.pallas import tpu as pltpu
from jax.experimental.pallas import tpu_sc as plsc

N_NODES = 10000
K = 32
D = 128
DW = D // 2

NC = 2
NS = 16
NW = NC * NS

N0 = 6672
PER0A = 416
PER0B = 432
PER1A = 208
PER1B = 208
PER_MAX = PER0B
CHUNK = 4
NBUF = 4
GROUP = CHUNK * NBUF
G0A = PER0A // GROUP
G0B = PER0B // GROUP
G1A = PER1A // GROUP
G1B = PER1B // GROUP
IDX_MAX = PER_MAX * K // D
NTBL = 10752
NIDX_ROWS = N_NODES * K // D + IDX_MAX


def _mm_body(x_ref, w_ref, z_ref):
    z_ref[...] = lax.dot_general(
        x_ref[...], w_ref[...],
        dimension_numbers=(((1,), (1,)), ((), ())),
        preferred_element_type=jnp.float32,
    ).astype(jnp.bfloat16)


def _matmul_packed(x, wp):
    blk = 448
    grid = 23
    return pl.pallas_call(
        _mm_body,
        grid=(grid,),
        in_specs=[
            pl.BlockSpec((blk, D), lambda i: (i, 0)),
            pl.BlockSpec((D, D), lambda i: (0, 0)),
        ],
        out_specs=pl.BlockSpec((blk, D), lambda i: (i, 0)),
        out_shape=jax.ShapeDtypeStruct((NTBL, D), jnp.bfloat16),
    )(x, wp)


_PERM = np.empty((D,), dtype=np.int32)
for _p in range(D // 32):
    for _k in range(16):
        _PERM[32 * _p + 2 * _k] = 32 * _p + _k
        _PERM[32 * _p + 2 * _k + 1] = 32 * _p + 16 + _k


def _unpack2(w):
    lo = lax.bitcast_convert_type(lax.shift_left(w, 16), jnp.float32)
    hi = lax.bitcast_convert_type(lax.bitwise_and(w, jnp.int32(-65536)),
                                  jnp.float32)
    return lo, hi


def _reduce_chunk(gat, own_v, out_v, t, urow):
    inv_k = jnp.float32(1.0 / K)
    zero = jnp.float32(0.0)
    for n in range(CHUNK):
        for p in range(DW // 16):
            sl = pl.ds(p * 16, 16)
            acc_lo, acc_hi = _unpack2(gat[n * K, sl])
            for j in range(1, K):
                lo, hi = _unpack2(gat[n * K + j, sl])
                acc_lo = acc_lo + lo
                acc_hi = acc_hi + hi
            own_lo, own_hi = _unpack2(own_v[t * CHUNK + n, sl])
            out_v[urow + n, pl.ds(32 * p, 16)] = jnp.maximum(
                own_lo - acc_lo * inv_k, zero)
            out_v[urow + n, pl.ds(32 * p + 16, 16)] = jnp.maximum(
                own_hi - acc_hi * inv_k, zero)


def _sc_body(z_hbm, nidx_hbm, out_hbm,
             idx_v, g0, g1, g2, g3, own_v, u_all,
             s0, s1, s2, s3, su, sem_own):
    c = lax.axis_index("c")
    s = lax.axis_index("s")
    node_base = jnp.where(c == 0, s * PER0A, N0 + s * PER1A)
    n_groups = jnp.where(c == 0, jnp.where(s < NS - 1, G0A, G0B),
                         jnp.where(s < NS - 1, G1A, G1B))
    n_chunks = n_groups * NBUF
    gbufs = (g0, g1, g2, g3)
    gsems = (s0, s1, s2, s3)

    own_cp = pltpu.async_copy(z_hbm.at[pl.ds(node_base, PER_MAX)], own_v,
                              sem_own)
    pltpu.sync_copy(nidx_hbm.at[pl.ds(node_base * K // D, IDX_MAX)], idx_v)

    def gather(t, g, sem):
        tc = jnp.minimum(t, n_chunks - 1)
        pltpu.async_copy(z_hbm.at[idx_v.at[tc]], g, sem)

    def wait_gather(g, sem):
        pltpu.make_async_copy(z_hbm.at[pl.ds(0, D)], g, sem).wait()

    for b in range(NBUF):
        gather(b, gbufs[b], gsems[b])
    own_cp.wait()

    def loop_body(grp, _):
        t0 = NBUF * grp
        @pl.when(grp >= 1)
        def _drain():
            pltpu.make_async_copy(
                u_all, out_hbm.at[pl.ds(0, GROUP)], su).wait()
        for b in range(NBUF):
            t = t0 + b
            wait_gather(gbufs[b], gsems[b])
            _reduce_chunk(gbufs[b], own_v, u_all, t, b * CHUNK)
            gather(t + NBUF, gbufs[b], gsems[b])
        pltpu.async_copy(
            u_all, out_hbm.at[pl.ds(node_base + t0 * CHUNK, GROUP)], su)
        return _

    lax.fori_loop(0, n_groups, loop_body, None)
    for b in range(NBUF):
        wait_gather(gbufs[b], gsems[b])
    pltpu.make_async_copy(u_all, out_hbm.at[pl.ds(0, GROUP)], su).wait()


_sc_call = functools.partial(
    pl.kernel,
    out_type=jax.ShapeDtypeStruct((N_NODES, D), jnp.float32),
    mesh=plsc.VectorSubcoreMesh(core_axis_name="c", subcore_axis_name="s"),
    compiler_params=pltpu.CompilerParams(use_tc_tiling_on_sc=False),
    scratch_types=[
        pltpu.VMEM((IDX_MAX, D), jnp.int32),
        pltpu.VMEM((D, DW), jnp.int32),
        pltpu.VMEM((D, DW), jnp.int32),
        pltpu.VMEM((D, DW), jnp.int32),
        pltpu.VMEM((D, DW), jnp.int32),
        pltpu.VMEM((PER_MAX, DW), jnp.int32),
        pltpu.VMEM((GROUP, D), jnp.float32),
        pltpu.SemaphoreType.DMA,
        pltpu.SemaphoreType.DMA,
        pltpu.SemaphoreType.DMA,
        pltpu.SemaphoreType.DMA,
        pltpu.SemaphoreType.DMA,
        pltpu.SemaphoreType.DMA,
    ],
)(_sc_body)


def kernel(X, neigh_idx, W):
    nidx2d = jnp.zeros((NIDX_ROWS, D), jnp.int32)
    nidx2d = nidx2d.at[:N_NODES * K // D].set(
        neigh_idx.reshape(N_NODES * K // D, D))
    wp = W[jnp.asarray(_PERM)]
    zb = _matmul_packed(X, wp)
    zi = lax.bitcast_convert_type(zb.reshape(NTBL, DW, 2), jnp.int32)
    return _sc_call(zi, nidx2d)
